# baseline (speedup 1.0000x reference)
import jax
import jax.numpy as jnp
from jax import lax
from jax.experimental import pallas as pl
from jax.experimental.pallas import tpu as pltpu
from jax.experimental.pallas import tpu_sc as plsc

_NC = 2
_NS = 16
_NW = _NC * _NS
_L = 16
_C = 384
_GRP = _C // _L
_ROWS = 4 * 64 * 64
_RPT = _ROWS // _NW
_RCH = 32
_NCH = _RPT // _RCH


def _sc_body(x0_hbm, x1_hbm, w1_hbm, w2_hbm, t_hbm, y1_hbm, y2_hbm,
             x0_v, x1_v, y1_v, y2_v, w1_v, w2_v, t_v):
    wid = lax.axis_index("s") * _NC + lax.axis_index("c")
    base = wid * _RPT
    pltpu.sync_copy(w1_hbm, w1_v)
    pltpu.sync_copy(w2_hbm, w2_v)
    pltpu.sync_copy(t_hbm, t_v)
    tv = t_v[...]
    m1 = [jnp.abs(w1_v[pl.ds(k * _L, _L)]) >= tv for k in range(_GRP)]
    m2 = [jnp.abs(w2_v[pl.ds(k * _L, _L)]) >= tv for k in range(_GRP)]

    def chunk(ci, carry):
        r0 = base + ci * _RCH
        pltpu.sync_copy(x0_hbm.at[pl.ds(r0, _RCH)], x0_v)
        pltpu.sync_copy(x1_hbm.at[pl.ds(r0, _RCH)], x1_v)

        def row(i, c2):
            for k in range(_GRP):
                s = pl.ds(k * _L, _L)
                a0 = x0_v[i, s]
                a1 = x1_v[i, s]
                y1_v[i, s] = jnp.where(m1[k], a0, a1)
                y2_v[i, s] = jnp.where(m2[k], a1, a0)
            return c2

        lax.fori_loop(0, _RCH, row, 0)
        pltpu.sync_copy(y1_v, y1_hbm.at[pl.ds(r0, _RCH)])
        pltpu.sync_copy(y2_v, y2_hbm.at[pl.ds(r0, _RCH)])
        return carry

    lax.fori_loop(0, _NCH, chunk, 0)


def kernel(x0, x1, bn1_weight, bn2_weight, bn_threshold):
    B, C, H, W = x0.shape
    x0t = jnp.transpose(x0, (0, 2, 3, 1)).reshape(B * H * W, C)
    x1t = jnp.transpose(x1, (0, 2, 3, 1)).reshape(B * H * W, C)
    tv = jnp.full((_L,), bn_threshold, jnp.float32)

    sck = pl.kernel(
        _sc_body,
        out_type=[jax.ShapeDtypeStruct((B * H * W, C), jnp.float32)] * 2,
        mesh=plsc.VectorSubcoreMesh(core_axis_name="c", subcore_axis_name="s"),
        scratch_types=[
            pltpu.VMEM((_RCH, _C), jnp.float32),
            pltpu.VMEM((_RCH, _C), jnp.float32),
            pltpu.VMEM((_RCH, _C), jnp.float32),
            pltpu.VMEM((_RCH, _C), jnp.float32),
            pltpu.VMEM((_C,), jnp.float32),
            pltpu.VMEM((_C,), jnp.float32),
            pltpu.VMEM((_L,), jnp.float32),
        ],
    )
    y1t, y2t = sck(x0t, x1t, bn1_weight, bn2_weight, tv)
    y1 = jnp.transpose(y1t.reshape(B, H, W, C), (0, 3, 1, 2))
    y2 = jnp.transpose(y2t.reshape(B, H, W, C), (0, 3, 1, 2))
    return y1, y2
